# baseline (speedup 1.0000x reference)
import functools

import jax
import jax.numpy as jnp
from jax import lax
from jax.experimental import pallas as pl
from jax.experimental.pallas import tpu as pltpu
from jax.experimental.pallas import tpu_sc as plsc

_NC = 2
_NS = 16


def _sc_gather(lab_hbm, c_hbm, out_hbm, idx_v, crow_v, gsem):
    cid = lax.axis_index("c")
    sid = lax.axis_index("s")
    wid = sid * _NC + cid

    @pl.when(wid == 0)
    def _go():
        pltpu.sync_copy(lab_hbm, idx_v)
        slab = idx_v[...][0]
        pltpu.async_copy(c_hbm.at[pl.ds(slab, 1), :], crow_v, gsem).wait()
        pltpu.sync_copy(crow_v, out_hbm)


def _tc_reduce(h_ref, sum_ref, col_ref, acc_ref):
    i = pl.program_id(0)
    n = pl.num_programs(0)

    @pl.when(i == 0)
    def _init():
        acc_ref[0] = 0.0

    x = h_ref[...]
    acc_ref[0] += jnp.sum(x * x)
    cs = jnp.sum(x, axis=0, keepdims=True)

    @pl.when(i == 0)
    def _first():
        col_ref[...] = cs

    @pl.when(i > 0)
    def _rest():
        col_ref[...] += cs

    @pl.when(i == n - 1)
    def _fin():
        sum_ref[0, 0] = acc_ref[0]


def kernel(h, labels, centers):
    B, D = h.shape
    lab = jnp.full((16,), labels, dtype=jnp.int32)
    mesh = plsc.VectorSubcoreMesh(core_axis_name="c", subcore_axis_name="s")
    row = pl.kernel(
        _sc_gather,
        out_type=jax.ShapeDtypeStruct((1, D), jnp.float32),
        mesh=mesh,
        scratch_types=[
            pltpu.VMEM((16,), jnp.int32),
            pltpu.VMEM((1, D), jnp.float32),
            pltpu.SemaphoreType.DMA,
        ],
    )(lab, centers)

    sumsq, colsum = pl.pallas_call(
        _tc_reduce,
        grid=(8,),
        in_specs=[pl.BlockSpec((B // 8, D), lambda i: (i, 0))],
        out_specs=[
            pl.BlockSpec((1, 1), lambda i: (0, 0), memory_space=pltpu.SMEM),
            pl.BlockSpec((1, D), lambda i: (0, 0)),
        ],
        out_shape=[
            jax.ShapeDtypeStruct((1, 1), jnp.float32),
            jax.ShapeDtypeStruct((1, D), jnp.float32),
        ],
        scratch_shapes=[pltpu.SMEM((1,), jnp.float32)],
    )(h)

    cross = jnp.sum(colsum * row)
    csq = jnp.sum(row * row)
    total = sumsq[0, 0] - 2.0 * cross + B * csq
    return (total / (B * D)).astype(jnp.float32)


# speedup vs baseline: 1.0097x; 1.0097x over previous
import functools

import jax
import jax.numpy as jnp
from jax import lax
from jax.experimental import pallas as pl
from jax.experimental.pallas import tpu as pltpu
from jax.experimental.pallas import tpu_sc as plsc

_NC = 2
_NS = 16
_NW = _NC * _NS


def _sc_body(rows_per_w, batch, h_hbm, lab_hbm, c_hbm, out_hbm,
             idx_v, xv, crow_v, pv, sem, gsem):
    cid = lax.axis_index("c")
    sid = lax.axis_index("s")
    wid = sid * _NC + cid
    base = wid * rows_per_w

    h_cp = pltpu.async_copy(h_hbm.at[pl.ds(base, rows_per_w), :], xv, sem)
    pltpu.sync_copy(lab_hbm, idx_v)
    slab = idx_v[...][0]
    pltpu.async_copy(c_hbm.at[pl.ds(slab, 1), :], crow_v, gsem).wait()
    h_cp.wait()

    zeros = jnp.zeros((16,), jnp.float32)

    def body(r, carry):
        c0, c1, c2, c3, sq = carry
        v0 = xv[r, pl.ds(0, 16)]
        v1 = xv[r, pl.ds(16, 16)]
        v2 = xv[r, pl.ds(32, 16)]
        v3 = xv[r, pl.ds(48, 16)]
        sq = sq + v0 * v0 + v1 * v1 + v2 * v2 + v3 * v3
        return (c0 + v0, c1 + v1, c2 + v2, c3 + v3, sq)

    c0, c1, c2, c3, sq = lax.fori_loop(
        0, rows_per_w, body, (zeros, zeros, zeros, zeros, zeros)
    )

    r0 = crow_v[0, pl.ds(0, 16)]
    r1 = crow_v[0, pl.ds(16, 16)]
    r2 = crow_v[0, pl.ds(32, 16)]
    r3 = crow_v[0, pl.ds(48, 16)]
    csq_share = jnp.float32(batch / _NW)
    partial = (sq - 2.0 * (c0 * r0 + c1 * r1 + c2 * r2 + c3 * r3)
               + csq_share * (r0 * r0 + r1 * r1 + r2 * r2 + r3 * r3))
    pv[...] = partial
    pltpu.sync_copy(pv, out_hbm.at[pl.ds(wid * 16, 16)])


def kernel(h, labels, centers):
    B, D = h.shape
    rows_per_w = B // _NW
    lab = jnp.full((16,), labels, dtype=jnp.int32)
    mesh = plsc.VectorSubcoreMesh(core_axis_name="c", subcore_axis_name="s")
    partials = pl.kernel(
        functools.partial(_sc_body, rows_per_w, float(B)),
        out_type=jax.ShapeDtypeStruct((_NW * 16,), jnp.float32),
        mesh=mesh,
        scratch_types=[
            pltpu.VMEM((16,), jnp.int32),
            pltpu.VMEM((rows_per_w, D), jnp.float32),
            pltpu.VMEM((1, D), jnp.float32),
            pltpu.VMEM((16,), jnp.float32),
            pltpu.SemaphoreType.DMA,
            pltpu.SemaphoreType.DMA,
        ],
    )(h, lab, centers)
    return (jnp.sum(partials) / (B * D)).astype(jnp.float32)
